# baseline (speedup 1.0000x reference)
import functools

import jax
import jax.numpy as jnp
import numpy as np
from jax.experimental import pallas as pl
from jax.experimental.pallas import tpu as pltpu

B = 2
N = 4096
D_IN = 128
NPOINT = 512
NSAMPLE = 32
D_OUT = 256
NHEAD = 8
NLAYERS = 4
EPS = 1e-5


_FR, _FC = 32, 128
_IR, _IC = 4, 128


def _fps_kernel(x_ref, y_ref, z_ref, idx_ref):
    x = x_ref[...]
    y = y_ref[...]
    z = z_ref[...]
    flat = (jax.lax.broadcasted_iota(jnp.int32, (B, _FR, _FC), 1) * _FC
            + jax.lax.broadcasted_iota(jnp.int32, (B, _FR, _FC), 2))
    oflat = (jax.lax.broadcasted_iota(jnp.int32, (B, _IR, _IC), 1) * _IC
             + jax.lax.broadcasted_iota(jnp.int32, (B, _IR, _IC), 2))

    def body(i, state):
        dmin, f, acc = state
        acc = jnp.where(oflat == i, f[:, None, None].astype(jnp.int32), acc)
        sel = flat == f[:, None, None]
        cx = jnp.sum(jnp.where(sel, x, 0.0), axis=(1, 2))
        cy = jnp.sum(jnp.where(sel, y, 0.0), axis=(1, 2))
        cz = jnp.sum(jnp.where(sel, z, 0.0), axis=(1, 2))
        dx = x - cx[:, None, None]
        dy = y - cy[:, None, None]
        dz = z - cz[:, None, None]
        dist = dx * dx + dy * dy + dz * dz
        dmin = jnp.minimum(dmin, dist)
        m = jnp.max(dmin, axis=(1, 2))
        fnew = jnp.min(jnp.where(dmin == m[:, None, None], flat, N), axis=(1, 2))
        return dmin, fnew, acc

    dmin0 = jnp.full((B, _FR, _FC), 1e10, jnp.float32)
    f0 = jnp.zeros((B,), jnp.int32)
    acc0 = jnp.zeros((B, _IR, _IC), jnp.int32)
    _, _, acc = jax.lax.fori_loop(0, NPOINT, body, (dmin0, f0, acc0))
    idx_ref[...] = acc


def _fps(xyzf, npoint):
    x = xyzf[..., 0].reshape(B, _FR, _FC)
    y = xyzf[..., 1].reshape(B, _FR, _FC)
    z = xyzf[..., 2].reshape(B, _FR, _FC)
    idx = pl.pallas_call(
        _fps_kernel,
        out_shape=jax.ShapeDtypeStruct((B, _IR, _IC), jnp.int32),
    )(x, y, z)
    return idx.reshape(B, NPOINT)


def _index_points(points, idx):
    b = jnp.arange(points.shape[0]).reshape((-1,) + (1,) * (idx.ndim - 1))
    return points[b, idx]


_FBIG = float(np.finfo(np.float32).max)


def _knn_kernel(xrow_ref, yrow_ref, zrow_ref, tab_ref, idxcol_ref, gi_ref, q_ref):
    xrow = xrow_ref[0]
    yrow = yrow_ref[0]
    zrow = zrow_ref[0]
    tab = tab_ref[0]
    idxcol = idxcol_ref[0]
    lane = jax.lax.broadcasted_iota(jnp.int32, (1, N), 1)

    oh = jnp.where(idxcol == lane, 1.0, 0.0)
    q = jnp.dot(oh, tab, preferred_element_type=jnp.float32,
                precision=jax.lax.Precision.HIGHEST)
    qx, qy, qz = q[:, 0:1], q[:, 1:2], q[:, 2:3]
    qn = qx * qx + qy * qy + qz * qz
    xn = xrow * xrow + yrow * yrow + zrow * zrow
    def _b(a):
        return a.astype(jnp.bfloat16).astype(jnp.float32)

    cross = _b(qx) * _b(xrow) + _b(qy) * _b(yrow) + _b(qz) * _b(zrow)
    d = (qn + xn) - 2.0 * cross

    kcol = jax.lax.broadcasted_iota(jnp.int32, (NPOINT, NSAMPLE), 1)

    def body(k, state):
        v, i, acc = state
        taken = (d < v) | ((d == v) & (lane <= i))
        mod = jnp.where(taken, _FBIG, d)
        m = jnp.min(mod, axis=-1, keepdims=True)
        sel = jnp.min(jnp.where(mod == m, lane, N), axis=-1, keepdims=True)
        acc = jnp.where(kcol == k, sel, acc)
        return m, sel, acc

    v0 = jnp.full((NPOINT, 1), -_FBIG, jnp.float32)
    i0 = jnp.full((NPOINT, 1), -1, jnp.int32)
    acc0 = jnp.zeros((NPOINT, NSAMPLE), jnp.int32)
    _, _, acc = jax.lax.fori_loop(0, NSAMPLE, body, (v0, i0, acc0))
    gi_ref[0] = acc
    q_ref[0] = q


def _knn_pallas(xyzf, fps_idx):
    xrow = xyzf[..., 0].reshape(B, 1, N)
    yrow = xyzf[..., 1].reshape(B, 1, N)
    zrow = xyzf[..., 2].reshape(B, 1, N)
    tab = jnp.concatenate([xyzf, jnp.zeros((B, N, 5), jnp.float32)], axis=-1)
    idxcol = fps_idx.reshape(B, NPOINT, 1)
    gi, q = pl.pallas_call(
        _knn_kernel,
        grid=(B,),
        in_specs=[
            pl.BlockSpec((1, 1, N), lambda b: (b, 0, 0)),
            pl.BlockSpec((1, 1, N), lambda b: (b, 0, 0)),
            pl.BlockSpec((1, 1, N), lambda b: (b, 0, 0)),
            pl.BlockSpec((1, N, 8), lambda b: (b, 0, 0)),
            pl.BlockSpec((1, NPOINT, 1), lambda b: (b, 0, 0)),
        ],
        out_specs=[
            pl.BlockSpec((1, NPOINT, NSAMPLE), lambda b: (b, 0, 0)),
            pl.BlockSpec((1, NPOINT, 8), lambda b: (b, 0, 0)),
        ],
        out_shape=[
            jax.ShapeDtypeStruct((B, NPOINT, NSAMPLE), jnp.int32),
            jax.ShapeDtypeStruct((B, NPOINT, 8), jnp.float32),
        ],
    )(xrow, yrow, zrow, tab, idxcol)
    return gi, q[..., :3]


_GB = 64
_TB = _GB * NSAMPLE
_NBLK = (B * NPOINT) // _GB
_SUB = 256
_NSB = _TB // _SUB
_HD = D_IN // NHEAD


def _bf(a):
    return a.astype(jnp.bfloat16)


def _dotbf(a, b, trans_b=False):
    dn = (((1,), (1 if trans_b else 0,)), ((), ()))
    return jax.lax.dot_general(_bf(a), _bf(b), dn,
                               preferred_element_type=jnp.float32)


def _ln_in_kernel(x, g, b):
    m = jnp.mean(x, -1, keepdims=True)
    v = jnp.mean((x - m) * (x - m), -1, keepdims=True)
    return (x - m) / jnp.sqrt(v + EPS) * g + b


def _tx_kernel(gfeat_ref, gxyz_ref,
               pe_w1t_ref, pe_b1_ref, bng_ref, bnb_ref, pe_w2t_ref, pe_b2_ref,
               wqkv_ref, bqkv_ref, wo_ref, bo_ref,
               wff1_ref, bff1_ref, wff2_ref, bff2_ref,
               ln1g_ref, ln1b_ref, ln2g_ref, ln2b_ref,
               pooled_ref, qkv_s, att_s):
    lane = jax.lax.broadcasted_iota(jnp.int32, (1, D_IN), 1)

    gx = gxyz_ref[...]
    h = _dotbf(gx, pe_w1t_ref[...]) + pe_b1_ref[...]
    h = h / np.float32(np.sqrt(1.0 + EPS)) * bng_ref[...] + bnb_ref[...]
    h = jnp.maximum(h, 0.0)
    pe = _dotbf(h, pe_w2t_ref[...]) + pe_b2_ref[...]
    x = gfeat_ref[...] + pe

    rsub = jax.lax.broadcasted_iota(jnp.int32, (_SUB, _SUB), 0)
    csub = jax.lax.broadcasted_iota(jnp.int32, (_SUB, _SUB), 1)
    blockmask = jnp.where((rsub // NSAMPLE) == (csub // NSAMPLE), 0.0, -1e30)

    for L in range(NLAYERS):
        qkv = _dotbf(x, wqkv_ref[L]) + bqkv_ref[L]
        qkv_s[...] = qkv

        def sb_body(sb, _):
            qs = qkv_s[pl.ds(sb * _SUB, _SUB), 0:D_IN]
            ks = qkv_s[pl.ds(sb * _SUB, _SUB), D_IN:2 * D_IN]
            vs = qkv_s[pl.ds(sb * _SUB, _SUB), 2 * D_IN:3 * D_IN]

            def h_body(hh, o_acc):
                maskh = jnp.where((lane // _HD) == hh, 1.0, 0.0)
                kh = ks * maskh
                S = _dotbf(qs, kh, trans_b=True) * np.float32(1.0 / np.sqrt(_HD))
                S = S + blockmask
                mx = jnp.max(S, axis=-1, keepdims=True)
                e = jnp.exp(S - mx)
                P = e / jnp.sum(e, axis=-1, keepdims=True)
                return o_acc + _dotbf(P, vs * maskh)

            o_sub = jax.lax.fori_loop(0, NHEAD, h_body,
                                      jnp.zeros((_SUB, D_IN), jnp.float32))
            att_s[pl.ds(sb * _SUB, _SUB), :] = o_sub
            return 0

        jax.lax.fori_loop(0, _NSB, sb_body, 0)
        o = _dotbf(att_s[...], wo_ref[L]) + bo_ref[L]
        x = _ln_in_kernel(x + o, ln1g_ref[L], ln1b_ref[L])
        hh = jnp.maximum(_dotbf(x, wff1_ref[L]) + bff1_ref[L], 0.0)
        ff = _dotbf(hh, wff2_ref[L]) + bff2_ref[L]
        x = _ln_in_kernel(x + ff, ln2g_ref[L], ln2b_ref[L])

    pooled_ref[...] = jnp.max(x.reshape(_GB, NSAMPLE, D_IN), axis=1)


def _tx_pallas(gfeat, gxyz, params):
    ls = params['layers']
    stk = lambda name: jnp.stack([l[name] for l in ls])
    wqkv = stk('w_qkv').transpose(0, 2, 1)
    bqkv = stk('b_qkv')[:, None, :]
    wo = stk('w_o').transpose(0, 2, 1)
    bo = stk('b_o')[:, None, :]
    wff1 = stk('w_ff1').transpose(0, 2, 1)
    bff1 = stk('b_ff1')[:, None, :]
    wff2 = stk('w_ff2').transpose(0, 2, 1)
    bff2 = stk('b_ff2')[:, None, :]
    ln1g = stk('ln1_g')[:, None, :]
    ln1b = stk('ln1_b')[:, None, :]
    ln2g = stk('ln2_g')[:, None, :]
    ln2b = stk('ln2_b')[:, None, :]
    pe_w1t = jnp.zeros((16, 64), jnp.float32).at[:3].set(params['pe_w1'].T)
    pe_b1 = params['pe_b1'][None, :]
    bng = params['pe_bn_g'][None, :]
    bnb = params['pe_bn_b'][None, :]
    pe_w2t = params['pe_w2'].T
    pe_b2 = params['pe_b2'][None, :]

    full = lambda shape: pl.BlockSpec(shape, lambda i: (0,) * len(shape))
    pooled = pl.pallas_call(
        _tx_kernel,
        grid=(_NBLK,),
        in_specs=[
            pl.BlockSpec((_TB, D_IN), lambda i: (i, 0)),
            pl.BlockSpec((_TB, 16), lambda i: (i, 0)),
            full((16, 64)), full((1, 64)), full((1, 64)), full((1, 64)),
            full((64, 128)), full((1, 128)),
            full((NLAYERS, 128, 384)), full((NLAYERS, 1, 384)),
            full((NLAYERS, 128, 128)), full((NLAYERS, 1, 128)),
            full((NLAYERS, 128, 256)), full((NLAYERS, 1, 256)),
            full((NLAYERS, 256, 128)), full((NLAYERS, 1, 128)),
            full((NLAYERS, 1, 128)), full((NLAYERS, 1, 128)),
            full((NLAYERS, 1, 128)), full((NLAYERS, 1, 128)),
        ],
        out_specs=pl.BlockSpec((_GB, D_IN), lambda i: (i, 0)),
        out_shape=jax.ShapeDtypeStruct((B * NPOINT, D_IN), jnp.float32),
        scratch_shapes=[
            pltpu.VMEM((_TB, 3 * D_IN), jnp.float32),
            pltpu.VMEM((_TB, D_IN), jnp.float32),
        ],
    )(gfeat, gxyz, pe_w1t, pe_b1, bng, bnb, pe_w2t, pe_b2,
      wqkv, bqkv, wo, bo, wff1, bff1, wff2, bff2, ln1g, ln1b, ln2g, ln2b)
    return pooled


def _build_wint_t():
    pos = np.arange(N, dtype=np.float32) * np.float32((NPOINT - 1) / (N - 1))
    lo = np.floor(pos).astype(np.int32)
    hi = np.minimum(lo + 1, NPOINT - 1)
    w = (pos - lo).astype(np.float32)
    m = np.zeros((N, NPOINT), np.float32)
    m[np.arange(N), lo] += (1.0 - w)
    m[np.arange(N), hi] += w
    return jnp.asarray(m)


def _fc_interp_kernel(pooled_ref, fcwt_ref, fcb_ref, wint_ref, out_ref):
    fco = _dotbf(pooled_ref[...], fcwt_ref[...]) + fcb_ref[...]
    up = jax.lax.dot_general(wint_ref[...], fco, (((1,), (0,)), ((), ())),
                             precision=jax.lax.Precision.HIGHEST,
                             preferred_element_type=jnp.float32)
    out_ref[0] = up


def _fc_interp_pallas(pooled, params):
    fcwt = params['fc_w'].T
    fcb = params['fc_b'][None, :]
    wint = _build_wint_t()
    up = pl.pallas_call(
        _fc_interp_kernel,
        grid=(B,),
        in_specs=[
            pl.BlockSpec((NPOINT, D_IN), lambda b: (b, 0)),
            pl.BlockSpec((D_IN, D_OUT), lambda b: (0, 0)),
            pl.BlockSpec((1, D_OUT), lambda b: (0, 0)),
            pl.BlockSpec((N, NPOINT), lambda b: (0, 0)),
        ],
        out_specs=pl.BlockSpec((1, N, D_OUT), lambda b: (b, 0, 0)),
        out_shape=jax.ShapeDtypeStruct((B, N, D_OUT), jnp.float32),
    )(pooled.reshape(B * NPOINT, D_IN), fcwt, fcb, wint)
    return up.transpose(0, 2, 1)


def _layer_norm(x, g, b):
    m = jnp.mean(x, -1, keepdims=True)
    v = jnp.var(x, -1, keepdims=True)
    return (x - m) / jnp.sqrt(v + EPS) * g + b


def _mha(x, p):
    S, T, D = x.shape
    hd = D // NHEAD
    qkv = jnp.einsum('std,ed->ste', x, p['w_qkv']) + p['b_qkv']
    q, k, v = jnp.split(qkv, 3, axis=-1)

    def heads(a):
        return a.reshape(S, T, NHEAD, hd).transpose(1, 2, 0, 3)

    q, k, v = heads(q), heads(k), heads(v)
    att = jax.nn.softmax(jnp.einsum('thsd,thud->thsu', q, k) / jnp.sqrt(float(hd)), axis=-1)
    o = jnp.einsum('thsu,thud->thsd', att, v).transpose(2, 0, 1, 3).reshape(S, T, D)
    return jnp.einsum('std,ed->ste', o, p['w_o']) + p['b_o']


def _encoder_layer(x, p):
    x = _layer_norm(x + _mha(x, p), p['ln1_g'], p['ln1_b'])
    h = jax.nn.relu(jnp.einsum('std,ed->ste', x, p['w_ff1']) + p['b_ff1'])
    ff = jnp.einsum('ste,de->std', h, p['w_ff2']) + p['b_ff2']
    return _layer_norm(x + ff, p['ln2_g'], p['ln2_b'])


def _conv1x1(x, w, b):
    return jnp.einsum('oc,bc...->bo...', w, x) + b.reshape((1, -1) + (1,) * (x.ndim - 2))


def _pe_net(gx, params):
    h = _conv1x1(gx, params['pe_w1'], params['pe_b1'])
    h = h / jnp.sqrt(1.0 + EPS) * params['pe_bn_g'].reshape(1, -1, 1, 1) + params['pe_bn_b'].reshape(1, -1, 1, 1)
    h = jax.nn.relu(h)
    return _conv1x1(h, params['pe_w2'], params['pe_b2'])


def _lin_interp(x, out_size):
    L = x.shape[-1]
    pos = jnp.arange(out_size) * ((L - 1) / (out_size - 1))
    lo = jnp.floor(pos).astype(jnp.int32)
    hi = jnp.minimum(lo + 1, L - 1)
    w = (pos - lo).astype(x.dtype)
    return x[..., lo] * (1.0 - w) + x[..., hi] * w


def _identity_kernel(x_ref, o_ref):
    o_ref[...] = x_ref[...]


def _pallas_identity(x):
    return pl.pallas_call(
        _identity_kernel,
        out_shape=jax.ShapeDtypeStruct(x.shape, x.dtype),
    )(x)


def kernel(xyz, features, params):
    xyzf = xyz.transpose(0, 2, 1)
    fps_idx = _fps(xyzf, NPOINT)
    group_idx, new_xyz = _knn_pallas(xyzf, fps_idx)

    gfeat_tok = _index_points(features.transpose(0, 2, 1), group_idx).reshape(-1, D_IN)
    gxyz3 = _index_points(xyzf, group_idx).reshape(-1, 3)
    gxyz_tok = jnp.concatenate(
        [gxyz3, jnp.zeros((B * NPOINT * NSAMPLE, 13), jnp.float32)], axis=-1)
    pooled = _tx_pallas(gfeat_tok, gxyz_tok, params)
    up = _fc_interp_pallas(pooled, params)
    return (new_xyz.transpose(0, 2, 1), up)


# speedup vs baseline: 2.0986x; 2.0986x over previous
import functools

import jax
import jax.numpy as jnp
import numpy as np
from jax.experimental import pallas as pl
from jax.experimental.pallas import tpu as pltpu

B = 2
N = 4096
D_IN = 128
NPOINT = 512
NSAMPLE = 32
D_OUT = 256
NHEAD = 8
NLAYERS = 4
EPS = 1e-5


_FR, _FC = 32, 128
_IR, _IC = 4, 128


def _fps_kernel(x_ref, y_ref, z_ref, idx_ref):
    x = x_ref[...]
    y = y_ref[...]
    z = z_ref[...]
    flat = (jax.lax.broadcasted_iota(jnp.int32, (B, _FR, _FC), 1) * _FC
            + jax.lax.broadcasted_iota(jnp.int32, (B, _FR, _FC), 2))
    oflat = (jax.lax.broadcasted_iota(jnp.int32, (B, _IR, _IC), 1) * _IC
             + jax.lax.broadcasted_iota(jnp.int32, (B, _IR, _IC), 2))

    def body(i, state):
        dmin, f, acc = state
        acc = jnp.where(oflat == i, f[:, None, None].astype(jnp.int32), acc)
        sel = flat == f[:, None, None]
        cx = jnp.sum(jnp.where(sel, x, 0.0), axis=(1, 2))
        cy = jnp.sum(jnp.where(sel, y, 0.0), axis=(1, 2))
        cz = jnp.sum(jnp.where(sel, z, 0.0), axis=(1, 2))
        dx = x - cx[:, None, None]
        dy = y - cy[:, None, None]
        dz = z - cz[:, None, None]
        dist = dx * dx + dy * dy + dz * dz
        dmin = jnp.minimum(dmin, dist)
        m = jnp.max(dmin, axis=(1, 2))
        fnew = jnp.min(jnp.where(dmin == m[:, None, None], flat, N), axis=(1, 2))
        return dmin, fnew, acc

    dmin0 = jnp.full((B, _FR, _FC), 1e10, jnp.float32)
    f0 = jnp.zeros((B,), jnp.int32)
    acc0 = jnp.zeros((B, _IR, _IC), jnp.int32)
    _, _, acc = jax.lax.fori_loop(0, NPOINT, body, (dmin0, f0, acc0))
    idx_ref[...] = acc


def _fps(xyzf, npoint):
    x = xyzf[..., 0].reshape(B, _FR, _FC)
    y = xyzf[..., 1].reshape(B, _FR, _FC)
    z = xyzf[..., 2].reshape(B, _FR, _FC)
    idx = pl.pallas_call(
        _fps_kernel,
        out_shape=jax.ShapeDtypeStruct((B, _IR, _IC), jnp.int32),
    )(x, y, z)
    return idx.reshape(B, NPOINT)


def _index_points(points, idx):
    b = jnp.arange(points.shape[0]).reshape((-1,) + (1,) * (idx.ndim - 1))
    return points[b, idx]


_FBIG = float(np.finfo(np.float32).max)


def _knn_kernel(xrow_ref, yrow_ref, zrow_ref, tab_ref, idxcol_ref, gi_ref, q_ref):
    xrow = xrow_ref[0]
    yrow = yrow_ref[0]
    zrow = zrow_ref[0]
    tab = tab_ref[0]
    idxcol = idxcol_ref[0]
    lane = jax.lax.broadcasted_iota(jnp.int32, (1, N), 1)

    oh = jnp.where(idxcol == lane, 1.0, 0.0)
    q = jnp.dot(oh, tab, preferred_element_type=jnp.float32,
                precision=jax.lax.Precision.HIGHEST)
    qx, qy, qz = q[:, 0:1], q[:, 1:2], q[:, 2:3]
    qn = qx * qx + qy * qy + qz * qz
    xn = xrow * xrow + yrow * yrow + zrow * zrow
    def _b(a):
        return a.astype(jnp.bfloat16).astype(jnp.float32)

    cross = _b(qx) * _b(xrow) + _b(qy) * _b(yrow) + _b(qz) * _b(zrow)
    d = (qn + xn) - 2.0 * cross

    kcol = jax.lax.broadcasted_iota(jnp.int32, (NPOINT, NSAMPLE), 1)

    def body(k, state):
        v, i, acc = state
        taken = (d < v) | ((d == v) & (lane <= i))
        mod = jnp.where(taken, _FBIG, d)
        m = jnp.min(mod, axis=-1, keepdims=True)
        sel = jnp.min(jnp.where(mod == m, lane, N), axis=-1, keepdims=True)
        acc = jnp.where(kcol == k, sel, acc)
        return m, sel, acc

    v0 = jnp.full((NPOINT, 1), -_FBIG, jnp.float32)
    i0 = jnp.full((NPOINT, 1), -1, jnp.int32)
    acc0 = jnp.zeros((NPOINT, NSAMPLE), jnp.int32)
    _, _, acc = jax.lax.fori_loop(0, NSAMPLE, body, (v0, i0, acc0))
    gi_ref[0] = acc
    q_ref[0] = q


def _knn_pallas(xyzf, fps_idx):
    xrow = xyzf[..., 0].reshape(B, 1, N)
    yrow = xyzf[..., 1].reshape(B, 1, N)
    zrow = xyzf[..., 2].reshape(B, 1, N)
    tab = jnp.concatenate([xyzf, jnp.zeros((B, N, 5), jnp.float32)], axis=-1)
    idxcol = fps_idx.reshape(B, NPOINT, 1)
    gi, q = pl.pallas_call(
        _knn_kernel,
        grid=(B,),
        in_specs=[
            pl.BlockSpec((1, 1, N), lambda b: (b, 0, 0)),
            pl.BlockSpec((1, 1, N), lambda b: (b, 0, 0)),
            pl.BlockSpec((1, 1, N), lambda b: (b, 0, 0)),
            pl.BlockSpec((1, N, 8), lambda b: (b, 0, 0)),
            pl.BlockSpec((1, NPOINT, 1), lambda b: (b, 0, 0)),
        ],
        out_specs=[
            pl.BlockSpec((1, NPOINT, NSAMPLE), lambda b: (b, 0, 0)),
            pl.BlockSpec((1, NPOINT, 8), lambda b: (b, 0, 0)),
        ],
        out_shape=[
            jax.ShapeDtypeStruct((B, NPOINT, NSAMPLE), jnp.int32),
            jax.ShapeDtypeStruct((B, NPOINT, 8), jnp.float32),
        ],
    )(xrow, yrow, zrow, tab, idxcol)
    return gi, q[..., :3]


_GB = 64
_TB = _GB * NSAMPLE
_NBLK = (B * NPOINT) // _GB
_SUB = 128
_NSB = _TB // _SUB
_HD = D_IN // NHEAD


def _bf(a):
    return a.astype(jnp.bfloat16)


def _dotbf(a, b, trans_b=False):
    dn = (((1,), (1 if trans_b else 0,)), ((), ()))
    return jax.lax.dot_general(_bf(a), _bf(b), dn,
                               preferred_element_type=jnp.float32)


def _ln_in_kernel(x, g, b):
    m = jnp.mean(x, -1, keepdims=True)
    v = jnp.mean((x - m) * (x - m), -1, keepdims=True)
    return (x - m) / jnp.sqrt(v + EPS) * g + b


def _tx_kernel(gfeat_ref, gxyz_ref,
               pe_w1t_ref, pe_b1_ref, bng_ref, bnb_ref, pe_w2t_ref, pe_b2_ref,
               wqkv_ref, bqkv_ref, wo_ref, bo_ref,
               wff1_ref, bff1_ref, wff2_ref, bff2_ref,
               ln1g_ref, ln1b_ref, ln2g_ref, ln2b_ref,
               pooled_ref):
    lane = jax.lax.broadcasted_iota(jnp.int32, (1, D_IN), 1)

    gx = gxyz_ref[...]
    h = _dotbf(gx, pe_w1t_ref[...]) + pe_b1_ref[...]
    h = h / np.float32(np.sqrt(1.0 + EPS)) * bng_ref[...] + bnb_ref[...]
    h = jnp.maximum(h, 0.0)
    pe = _dotbf(h, pe_w2t_ref[...]) + pe_b2_ref[...]
    x = gfeat_ref[...] + pe

    masks = [jnp.where((lane // _HD) == h, 1.0, 0.0) for h in range(NHEAD)]
    rstk = jax.lax.broadcasted_iota(jnp.int32, (NHEAD * _SUB, _SUB), 0)
    cstk = jax.lax.broadcasted_iota(jnp.int32, (NHEAD * _SUB, _SUB), 1)
    blockmask = jnp.where(((rstk % _SUB) // NSAMPLE) == (cstk // NSAMPLE),
                          0.0, -1e30)

    for L in range(NLAYERS):
        qkv = _dotbf(x, wqkv_ref[L]) + bqkv_ref[L]
        q, k, v = qkv[:, :D_IN], qkv[:, D_IN:2 * D_IN], qkv[:, 2 * D_IN:]

        o_parts = []
        for sb in range(_NSB):
            qs = q[sb * _SUB:(sb + 1) * _SUB]
            ks = k[sb * _SUB:(sb + 1) * _SUB]
            vs = v[sb * _SUB:(sb + 1) * _SUB]
            qstk = jnp.concatenate([qs * m for m in masks], axis=0)
            S = _dotbf(qstk, ks, trans_b=True) * np.float32(1.0 / np.sqrt(_HD))
            S = S + blockmask
            mx = jnp.max(S, axis=-1, keepdims=True)
            e = jnp.exp(S - mx)
            P = e / jnp.sum(e, axis=-1, keepdims=True)
            ostk = _dotbf(P, vs)
            o_sub = ostk[0:_SUB] * masks[0]
            for h in range(1, NHEAD):
                o_sub = o_sub + ostk[h * _SUB:(h + 1) * _SUB] * masks[h]
            o_parts.append(o_sub)
        o = jnp.concatenate(o_parts, axis=0)
        o = _dotbf(o, wo_ref[L]) + bo_ref[L]
        x = _ln_in_kernel(x + o, ln1g_ref[L], ln1b_ref[L])
        hh = jnp.maximum(_dotbf(x, wff1_ref[L]) + bff1_ref[L], 0.0)
        ff = _dotbf(hh, wff2_ref[L]) + bff2_ref[L]
        x = _ln_in_kernel(x + ff, ln2g_ref[L], ln2b_ref[L])

    pooled_ref[...] = jnp.max(x.reshape(_GB, NSAMPLE, D_IN), axis=1)


def _tx_pallas(gfeat, gxyz, params):
    ls = params['layers']
    stk = lambda name: jnp.stack([l[name] for l in ls])
    wqkv = stk('w_qkv').transpose(0, 2, 1)
    bqkv = stk('b_qkv')[:, None, :]
    wo = stk('w_o').transpose(0, 2, 1)
    bo = stk('b_o')[:, None, :]
    wff1 = stk('w_ff1').transpose(0, 2, 1)
    bff1 = stk('b_ff1')[:, None, :]
    wff2 = stk('w_ff2').transpose(0, 2, 1)
    bff2 = stk('b_ff2')[:, None, :]
    ln1g = stk('ln1_g')[:, None, :]
    ln1b = stk('ln1_b')[:, None, :]
    ln2g = stk('ln2_g')[:, None, :]
    ln2b = stk('ln2_b')[:, None, :]
    pe_w1t = jnp.zeros((16, 64), jnp.float32).at[:3].set(params['pe_w1'].T)
    pe_b1 = params['pe_b1'][None, :]
    bng = params['pe_bn_g'][None, :]
    bnb = params['pe_bn_b'][None, :]
    pe_w2t = params['pe_w2'].T
    pe_b2 = params['pe_b2'][None, :]

    full = lambda shape: pl.BlockSpec(shape, lambda i: (0,) * len(shape))
    pooled = pl.pallas_call(
        _tx_kernel,
        grid=(_NBLK,),
        in_specs=[
            pl.BlockSpec((_TB, D_IN), lambda i: (i, 0)),
            pl.BlockSpec((_TB, 16), lambda i: (i, 0)),
            full((16, 64)), full((1, 64)), full((1, 64)), full((1, 64)),
            full((64, 128)), full((1, 128)),
            full((NLAYERS, 128, 384)), full((NLAYERS, 1, 384)),
            full((NLAYERS, 128, 128)), full((NLAYERS, 1, 128)),
            full((NLAYERS, 128, 256)), full((NLAYERS, 1, 256)),
            full((NLAYERS, 256, 128)), full((NLAYERS, 1, 128)),
            full((NLAYERS, 1, 128)), full((NLAYERS, 1, 128)),
            full((NLAYERS, 1, 128)), full((NLAYERS, 1, 128)),
        ],
        out_specs=pl.BlockSpec((_GB, D_IN), lambda i: (i, 0)),
        out_shape=jax.ShapeDtypeStruct((B * NPOINT, D_IN), jnp.float32),
    )(gfeat, gxyz, pe_w1t, pe_b1, bng, bnb, pe_w2t, pe_b2,
      wqkv, bqkv, wo, bo, wff1, bff1, wff2, bff2, ln1g, ln1b, ln2g, ln2b)
    return pooled


def _build_wint_t():
    pos = np.arange(N, dtype=np.float32) * np.float32((NPOINT - 1) / (N - 1))
    lo = np.floor(pos).astype(np.int32)
    hi = np.minimum(lo + 1, NPOINT - 1)
    w = (pos - lo).astype(np.float32)
    m = np.zeros((N, NPOINT), np.float32)
    m[np.arange(N), lo] += (1.0 - w)
    m[np.arange(N), hi] += w
    return jnp.asarray(m)


def _fc_interp_kernel(pooled_ref, fcwt_ref, fcb_ref, wint_ref, out_ref):
    fco = _dotbf(pooled_ref[...], fcwt_ref[...]) + fcb_ref[...]
    up = jax.lax.dot_general(wint_ref[...], fco, (((1,), (0,)), ((), ())),
                             precision=jax.lax.Precision.HIGHEST,
                             preferred_element_type=jnp.float32)
    out_ref[0] = up


def _fc_interp_pallas(pooled, params):
    fcwt = params['fc_w'].T
    fcb = params['fc_b'][None, :]
    wint = _build_wint_t()
    up = pl.pallas_call(
        _fc_interp_kernel,
        grid=(B,),
        in_specs=[
            pl.BlockSpec((NPOINT, D_IN), lambda b: (b, 0)),
            pl.BlockSpec((D_IN, D_OUT), lambda b: (0, 0)),
            pl.BlockSpec((1, D_OUT), lambda b: (0, 0)),
            pl.BlockSpec((N, NPOINT), lambda b: (0, 0)),
        ],
        out_specs=pl.BlockSpec((1, N, D_OUT), lambda b: (b, 0, 0)),
        out_shape=jax.ShapeDtypeStruct((B, N, D_OUT), jnp.float32),
    )(pooled.reshape(B * NPOINT, D_IN), fcwt, fcb, wint)
    return up.transpose(0, 2, 1)


def _layer_norm(x, g, b):
    m = jnp.mean(x, -1, keepdims=True)
    v = jnp.var(x, -1, keepdims=True)
    return (x - m) / jnp.sqrt(v + EPS) * g + b


def _mha(x, p):
    S, T, D = x.shape
    hd = D // NHEAD
    qkv = jnp.einsum('std,ed->ste', x, p['w_qkv']) + p['b_qkv']
    q, k, v = jnp.split(qkv, 3, axis=-1)

    def heads(a):
        return a.reshape(S, T, NHEAD, hd).transpose(1, 2, 0, 3)

    q, k, v = heads(q), heads(k), heads(v)
    att = jax.nn.softmax(jnp.einsum('thsd,thud->thsu', q, k) / jnp.sqrt(float(hd)), axis=-1)
    o = jnp.einsum('thsu,thud->thsd', att, v).transpose(2, 0, 1, 3).reshape(S, T, D)
    return jnp.einsum('std,ed->ste', o, p['w_o']) + p['b_o']


def _encoder_layer(x, p):
    x = _layer_norm(x + _mha(x, p), p['ln1_g'], p['ln1_b'])
    h = jax.nn.relu(jnp.einsum('std,ed->ste', x, p['w_ff1']) + p['b_ff1'])
    ff = jnp.einsum('ste,de->std', h, p['w_ff2']) + p['b_ff2']
    return _layer_norm(x + ff, p['ln2_g'], p['ln2_b'])


def _conv1x1(x, w, b):
    return jnp.einsum('oc,bc...->bo...', w, x) + b.reshape((1, -1) + (1,) * (x.ndim - 2))


def _pe_net(gx, params):
    h = _conv1x1(gx, params['pe_w1'], params['pe_b1'])
    h = h / jnp.sqrt(1.0 + EPS) * params['pe_bn_g'].reshape(1, -1, 1, 1) + params['pe_bn_b'].reshape(1, -1, 1, 1)
    h = jax.nn.relu(h)
    return _conv1x1(h, params['pe_w2'], params['pe_b2'])


def _lin_interp(x, out_size):
    L = x.shape[-1]
    pos = jnp.arange(out_size) * ((L - 1) / (out_size - 1))
    lo = jnp.floor(pos).astype(jnp.int32)
    hi = jnp.minimum(lo + 1, L - 1)
    w = (pos - lo).astype(x.dtype)
    return x[..., lo] * (1.0 - w) + x[..., hi] * w


def _identity_kernel(x_ref, o_ref):
    o_ref[...] = x_ref[...]


def _pallas_identity(x):
    return pl.pallas_call(
        _identity_kernel,
        out_shape=jax.ShapeDtypeStruct(x.shape, x.dtype),
    )(x)


def kernel(xyz, features, params):
    xyzf = xyz.transpose(0, 2, 1)
    fps_idx = _fps(xyzf, NPOINT)
    group_idx, new_xyz = _knn_pallas(xyzf, fps_idx)

    gfeat_tok = _index_points(features.transpose(0, 2, 1), group_idx).reshape(-1, D_IN)
    gxyz3 = _index_points(xyzf, group_idx).reshape(-1, 3)
    gxyz_tok = jnp.concatenate(
        [gxyz3, jnp.zeros((B * NPOINT * NSAMPLE, 13), jnp.float32)], axis=-1)
    pooled = _tx_pallas(gfeat_tok, gxyz_tok, params)
    up = _fc_interp_pallas(pooled, params)
    return (new_xyz.transpose(0, 2, 1), up)


# speedup vs baseline: 3.9300x; 1.8727x over previous
import functools

import jax
import jax.numpy as jnp
import numpy as np
from jax import lax
from jax.experimental import pallas as pl
from jax.experimental.pallas import tpu as pltpu
from jax.experimental.pallas import tpu_sc as plsc

B = 2
N = 4096
D_IN = 128
NPOINT = 512
NSAMPLE = 32
D_OUT = 256
NHEAD = 8
NLAYERS = 4
EPS = 1e-5


_FR, _FC = 32, 128
_IR, _IC = 4, 128


def _fps_kernel(x_ref, y_ref, z_ref, idx_ref):
    x = x_ref[...]
    y = y_ref[...]
    z = z_ref[...]
    flat = (jax.lax.broadcasted_iota(jnp.int32, (B, _FR, _FC), 1) * _FC
            + jax.lax.broadcasted_iota(jnp.int32, (B, _FR, _FC), 2))
    oflat = (jax.lax.broadcasted_iota(jnp.int32, (B, _IR, _IC), 1) * _IC
             + jax.lax.broadcasted_iota(jnp.int32, (B, _IR, _IC), 2))

    def body(i, state):
        dmin, f, acc = state
        acc = jnp.where(oflat == i, f[:, None, None].astype(jnp.int32), acc)
        sel = flat == f[:, None, None]
        cx = jnp.sum(jnp.where(sel, x, 0.0), axis=(1, 2))
        cy = jnp.sum(jnp.where(sel, y, 0.0), axis=(1, 2))
        cz = jnp.sum(jnp.where(sel, z, 0.0), axis=(1, 2))
        dx = x - cx[:, None, None]
        dy = y - cy[:, None, None]
        dz = z - cz[:, None, None]
        dist = dx * dx + dy * dy + dz * dz
        dmin = jnp.minimum(dmin, dist)
        m = jnp.max(dmin, axis=(1, 2))
        fnew = jnp.min(jnp.where(dmin == m[:, None, None], flat, N), axis=(1, 2))
        return dmin, fnew, acc

    dmin0 = jnp.full((B, _FR, _FC), 1e10, jnp.float32)
    f0 = jnp.zeros((B,), jnp.int32)
    acc0 = jnp.zeros((B, _IR, _IC), jnp.int32)
    _, _, acc = jax.lax.fori_loop(0, NPOINT, body, (dmin0, f0, acc0))
    idx_ref[...] = acc


def _fps(xyzf, npoint):
    x = xyzf[..., 0].reshape(B, _FR, _FC)
    y = xyzf[..., 1].reshape(B, _FR, _FC)
    z = xyzf[..., 2].reshape(B, _FR, _FC)
    idx = pl.pallas_call(
        _fps_kernel,
        out_shape=jax.ShapeDtypeStruct((B, _IR, _IC), jnp.int32),
    )(x, y, z)
    return idx.reshape(B, NPOINT)


def _index_points(points, idx):
    b = jnp.arange(points.shape[0]).reshape((-1,) + (1,) * (idx.ndim - 1))
    return points[b, idx]


_FBIG = float(np.finfo(np.float32).max)


def _knn_kernel(xrow_ref, yrow_ref, zrow_ref, tab_ref, idxcol_ref, gi_ref, q_ref):
    xrow = xrow_ref[0]
    yrow = yrow_ref[0]
    zrow = zrow_ref[0]
    tab = tab_ref[0]
    idxcol = idxcol_ref[0]
    lane = jax.lax.broadcasted_iota(jnp.int32, (1, N), 1)

    oh = jnp.where(idxcol == lane, 1.0, 0.0)
    q = jnp.dot(oh, tab, preferred_element_type=jnp.float32,
                precision=jax.lax.Precision.HIGHEST)
    qx, qy, qz = q[:, 0:1], q[:, 1:2], q[:, 2:3]
    qn = qx * qx + qy * qy + qz * qz
    xn = xrow * xrow + yrow * yrow + zrow * zrow
    def _b(a):
        return a.astype(jnp.bfloat16).astype(jnp.float32)

    cross = _b(qx) * _b(xrow) + _b(qy) * _b(yrow) + _b(qz) * _b(zrow)
    d = (qn + xn) - 2.0 * cross

    kcol = jax.lax.broadcasted_iota(jnp.int32, (NPOINT, NSAMPLE), 1)

    def body(k, state):
        v, i, acc = state
        taken = (d < v) | ((d == v) & (lane <= i))
        mod = jnp.where(taken, _FBIG, d)
        m = jnp.min(mod, axis=-1, keepdims=True)
        sel = jnp.min(jnp.where(mod == m, lane, N), axis=-1, keepdims=True)
        acc = jnp.where(kcol == k, sel, acc)
        return m, sel, acc

    v0 = jnp.full((NPOINT, 1), -_FBIG, jnp.float32)
    i0 = jnp.full((NPOINT, 1), -1, jnp.int32)
    acc0 = jnp.zeros((NPOINT, NSAMPLE), jnp.int32)
    _, _, acc = jax.lax.fori_loop(0, NSAMPLE, body, (v0, i0, acc0))
    gi_ref[0] = acc
    q_ref[0] = q


def _knn_pallas(xyzf, fps_idx):
    xrow = xyzf[..., 0].reshape(B, 1, N)
    yrow = xyzf[..., 1].reshape(B, 1, N)
    zrow = xyzf[..., 2].reshape(B, 1, N)
    tab = jnp.concatenate([xyzf, jnp.zeros((B, N, 5), jnp.float32)], axis=-1)
    idxcol = fps_idx.reshape(B, NPOINT, 1)
    gi, q = pl.pallas_call(
        _knn_kernel,
        grid=(B,),
        in_specs=[
            pl.BlockSpec((1, 1, N), lambda b: (b, 0, 0)),
            pl.BlockSpec((1, 1, N), lambda b: (b, 0, 0)),
            pl.BlockSpec((1, 1, N), lambda b: (b, 0, 0)),
            pl.BlockSpec((1, N, 8), lambda b: (b, 0, 0)),
            pl.BlockSpec((1, NPOINT, 1), lambda b: (b, 0, 0)),
        ],
        out_specs=[
            pl.BlockSpec((1, NPOINT, NSAMPLE), lambda b: (b, 0, 0)),
            pl.BlockSpec((1, NPOINT, 8), lambda b: (b, 0, 0)),
        ],
        out_shape=[
            jax.ShapeDtypeStruct((B, NPOINT, NSAMPLE), jnp.int32),
            jax.ShapeDtypeStruct((B, NPOINT, 8), jnp.float32),
        ],
    )(xrow, yrow, zrow, tab, idxcol)
    return gi, q[..., :3]


_NTOK = B * NPOINT * NSAMPLE
_NW = 32
_ROWS_W = _NTOK // _NW
_CHUNK = 128
_NCH = _ROWS_W // _CHUNK


def _sc_gather_kernel(tab_hbm, idx_hbm, out_hbm, idx_v, rows_v, sem):
    wid = lax.axis_index("s") * 2 + lax.axis_index("c")
    for c in range(_NCH):
        base = wid * _ROWS_W + c * _CHUNK
        pltpu.sync_copy(idx_hbm.at[pl.ds(base, _CHUNK)], idx_v)
        pltpu.async_copy(tab_hbm.at[idx_v], rows_v, sem).wait()
        pltpu.sync_copy(rows_v, out_hbm.at[pl.ds(base, _CHUNK)])


def _sc_gather(tab, flat_idx):
    mesh = plsc.VectorSubcoreMesh(core_axis_name="c", subcore_axis_name="s")
    k = functools.partial(
        pl.kernel, mesh=mesh,
        out_type=jax.ShapeDtypeStruct((_NTOK, D_IN), jnp.float32),
        scratch_types=[
            pltpu.VMEM((_CHUNK,), jnp.int32),
            pltpu.VMEM((_CHUNK, D_IN), jnp.float32),
            pltpu.SemaphoreType.DMA,
        ],
    )(_sc_gather_kernel)
    return k(tab, flat_idx)


def _petab_kernel(gx_ref, feat_ref,
                  pe_w1t_ref, pe_b1_ref, bng_ref, bnb_ref,
                  pe_w2t_ref, pe_b2_ref, out_ref):
    gx = gx_ref[...]
    h = _dotbf(gx, pe_w1t_ref[...]) + pe_b1_ref[...]
    h = h / np.float32(np.sqrt(1.0 + EPS)) * bng_ref[...] + bnb_ref[...]
    h = jnp.maximum(h, 0.0)
    pe = _dotbf(h, pe_w2t_ref[...]) + pe_b2_ref[...]
    out_ref[...] = feat_ref[...] + pe


def _petab_pallas(xyz_tab, feat_tab, params):
    pe_w1t = jnp.zeros((16, 64), jnp.float32).at[:3].set(params['pe_w1'].T)
    pe_b1 = params['pe_b1'][None, :]
    bng = params['pe_bn_g'][None, :]
    bnb = params['pe_bn_b'][None, :]
    pe_w2t = params['pe_w2'].T
    pe_b2 = params['pe_b2'][None, :]
    blk = 2048
    full = lambda shape: pl.BlockSpec(shape, lambda i: (0,) * len(shape))
    return pl.pallas_call(
        _petab_kernel,
        grid=((B * N) // blk,),
        in_specs=[
            pl.BlockSpec((blk, 16), lambda i: (i, 0)),
            pl.BlockSpec((blk, D_IN), lambda i: (i, 0)),
            full((16, 64)), full((1, 64)), full((1, 64)), full((1, 64)),
            full((64, 128)), full((1, 128)),
        ],
        out_specs=pl.BlockSpec((blk, D_IN), lambda i: (i, 0)),
        out_shape=jax.ShapeDtypeStruct((B * N, D_IN), jnp.float32),
    )(xyz_tab, feat_tab, pe_w1t, pe_b1, bng, bnb, pe_w2t, pe_b2)


_GB = 64
_TB = _GB * NSAMPLE
_NBLK = (B * NPOINT) // _GB
_SUB = 128
_NSB = _TB // _SUB
_HD = D_IN // NHEAD


def _bf(a):
    return a.astype(jnp.bfloat16)


def _dotbf(a, b, trans_b=False):
    dn = (((1,), (1 if trans_b else 0,)), ((), ()))
    return jax.lax.dot_general(_bf(a), _bf(b), dn,
                               preferred_element_type=jnp.float32)


def _ln_in_kernel(x, g, b):
    m = jnp.mean(x, -1, keepdims=True)
    v = jnp.mean((x - m) * (x - m), -1, keepdims=True)
    return (x - m) / jnp.sqrt(v + EPS) * g + b


def _tx_kernel(gfeat_ref,
               wqkv_ref, bqkv_ref, wo_ref, bo_ref,
               wff1_ref, bff1_ref, wff2_ref, bff2_ref,
               ln1g_ref, ln1b_ref, ln2g_ref, ln2b_ref,
               pooled_ref):
    lane = jax.lax.broadcasted_iota(jnp.int32, (1, D_IN), 1)
    x = gfeat_ref[...]

    masks = [jnp.where((lane // _HD) == h, 1.0, 0.0) for h in range(NHEAD)]
    rstk = jax.lax.broadcasted_iota(jnp.int32, (NHEAD * _SUB, _SUB), 0)
    cstk = jax.lax.broadcasted_iota(jnp.int32, (NHEAD * _SUB, _SUB), 1)
    blockmask = jnp.where(((rstk % _SUB) // NSAMPLE) == (cstk // NSAMPLE),
                          0.0, -1e30)

    for L in range(NLAYERS):
        qkv = _dotbf(x, wqkv_ref[L]) + bqkv_ref[L]
        q, k, v = qkv[:, :D_IN], qkv[:, D_IN:2 * D_IN], qkv[:, 2 * D_IN:]

        o_parts = []
        for sb in range(_NSB):
            qs = q[sb * _SUB:(sb + 1) * _SUB]
            ks = k[sb * _SUB:(sb + 1) * _SUB]
            vs = v[sb * _SUB:(sb + 1) * _SUB]
            qstk = jnp.concatenate([qs * m for m in masks], axis=0)
            S = _dotbf(qstk, ks, trans_b=True) * np.float32(1.0 / np.sqrt(_HD))
            S = S + blockmask
            mx = jnp.max(S, axis=-1, keepdims=True)
            e = jnp.exp(S - mx)
            P = e / jnp.sum(e, axis=-1, keepdims=True)
            ostk = _dotbf(P, vs)
            o_sub = ostk[0:_SUB] * masks[0]
            for h in range(1, NHEAD):
                o_sub = o_sub + ostk[h * _SUB:(h + 1) * _SUB] * masks[h]
            o_parts.append(o_sub)
        o = jnp.concatenate(o_parts, axis=0)
        o = _dotbf(o, wo_ref[L]) + bo_ref[L]
        x = _ln_in_kernel(x + o, ln1g_ref[L], ln1b_ref[L])
        hh = jnp.maximum(_dotbf(x, wff1_ref[L]) + bff1_ref[L], 0.0)
        ff = _dotbf(hh, wff2_ref[L]) + bff2_ref[L]
        x = _ln_in_kernel(x + ff, ln2g_ref[L], ln2b_ref[L])

    pooled_ref[...] = jnp.max(x.reshape(_GB, NSAMPLE, D_IN), axis=1)


def _tx_pallas(gfeat, params):
    ls = params['layers']
    stk = lambda name: jnp.stack([l[name] for l in ls])
    wqkv = stk('w_qkv').transpose(0, 2, 1)
    bqkv = stk('b_qkv')[:, None, :]
    wo = stk('w_o').transpose(0, 2, 1)
    bo = stk('b_o')[:, None, :]
    wff1 = stk('w_ff1').transpose(0, 2, 1)
    bff1 = stk('b_ff1')[:, None, :]
    wff2 = stk('w_ff2').transpose(0, 2, 1)
    bff2 = stk('b_ff2')[:, None, :]
    ln1g = stk('ln1_g')[:, None, :]
    ln1b = stk('ln1_b')[:, None, :]
    ln2g = stk('ln2_g')[:, None, :]
    ln2b = stk('ln2_b')[:, None, :]

    full = lambda shape: pl.BlockSpec(shape, lambda i: (0,) * len(shape))
    pooled = pl.pallas_call(
        _tx_kernel,
        grid=(_NBLK,),
        in_specs=[
            pl.BlockSpec((_TB, D_IN), lambda i: (i, 0)),
            full((NLAYERS, 128, 384)), full((NLAYERS, 1, 384)),
            full((NLAYERS, 128, 128)), full((NLAYERS, 1, 128)),
            full((NLAYERS, 128, 256)), full((NLAYERS, 1, 256)),
            full((NLAYERS, 256, 128)), full((NLAYERS, 1, 128)),
            full((NLAYERS, 1, 128)), full((NLAYERS, 1, 128)),
            full((NLAYERS, 1, 128)), full((NLAYERS, 1, 128)),
        ],
        out_specs=pl.BlockSpec((_GB, D_IN), lambda i: (i, 0)),
        out_shape=jax.ShapeDtypeStruct((B * NPOINT, D_IN), jnp.float32),
    )(gfeat, wqkv, bqkv, wo, bo, wff1, bff1, wff2, bff2,
      ln1g, ln1b, ln2g, ln2b)
    return pooled


def _build_wint_t():
    pos = np.arange(N, dtype=np.float32) * np.float32((NPOINT - 1) / (N - 1))
    lo = np.floor(pos).astype(np.int32)
    hi = np.minimum(lo + 1, NPOINT - 1)
    w = (pos - lo).astype(np.float32)
    m = np.zeros((N, NPOINT), np.float32)
    m[np.arange(N), lo] += (1.0 - w)
    m[np.arange(N), hi] += w
    return jnp.asarray(m)


def _fc_interp_kernel(pooled_ref, fcwt_ref, fcb_ref, wint_ref, out_ref):
    fco = _dotbf(pooled_ref[...], fcwt_ref[...]) + fcb_ref[...]
    up = jax.lax.dot_general(wint_ref[...], fco, (((1,), (0,)), ((), ())),
                             precision=jax.lax.Precision.HIGHEST,
                             preferred_element_type=jnp.float32)
    out_ref[0] = up


def _fc_interp_pallas(pooled, params):
    fcwt = params['fc_w'].T
    fcb = params['fc_b'][None, :]
    wint = _build_wint_t()
    up = pl.pallas_call(
        _fc_interp_kernel,
        grid=(B,),
        in_specs=[
            pl.BlockSpec((NPOINT, D_IN), lambda b: (b, 0)),
            pl.BlockSpec((D_IN, D_OUT), lambda b: (0, 0)),
            pl.BlockSpec((1, D_OUT), lambda b: (0, 0)),
            pl.BlockSpec((N, NPOINT), lambda b: (0, 0)),
        ],
        out_specs=pl.BlockSpec((1, N, D_OUT), lambda b: (b, 0, 0)),
        out_shape=jax.ShapeDtypeStruct((B, N, D_OUT), jnp.float32),
    )(pooled.reshape(B * NPOINT, D_IN), fcwt, fcb, wint)
    return up.transpose(0, 2, 1)


def _layer_norm(x, g, b):
    m = jnp.mean(x, -1, keepdims=True)
    v = jnp.var(x, -1, keepdims=True)
    return (x - m) / jnp.sqrt(v + EPS) * g + b


def _mha(x, p):
    S, T, D = x.shape
    hd = D // NHEAD
    qkv = jnp.einsum('std,ed->ste', x, p['w_qkv']) + p['b_qkv']
    q, k, v = jnp.split(qkv, 3, axis=-1)

    def heads(a):
        return a.reshape(S, T, NHEAD, hd).transpose(1, 2, 0, 3)

    q, k, v = heads(q), heads(k), heads(v)
    att = jax.nn.softmax(jnp.einsum('thsd,thud->thsu', q, k) / jnp.sqrt(float(hd)), axis=-1)
    o = jnp.einsum('thsu,thud->thsd', att, v).transpose(2, 0, 1, 3).reshape(S, T, D)
    return jnp.einsum('std,ed->ste', o, p['w_o']) + p['b_o']


def _encoder_layer(x, p):
    x = _layer_norm(x + _mha(x, p), p['ln1_g'], p['ln1_b'])
    h = jax.nn.relu(jnp.einsum('std,ed->ste', x, p['w_ff1']) + p['b_ff1'])
    ff = jnp.einsum('ste,de->std', h, p['w_ff2']) + p['b_ff2']
    return _layer_norm(x + ff, p['ln2_g'], p['ln2_b'])


def _conv1x1(x, w, b):
    return jnp.einsum('oc,bc...->bo...', w, x) + b.reshape((1, -1) + (1,) * (x.ndim - 2))


def _pe_net(gx, params):
    h = _conv1x1(gx, params['pe_w1'], params['pe_b1'])
    h = h / jnp.sqrt(1.0 + EPS) * params['pe_bn_g'].reshape(1, -1, 1, 1) + params['pe_bn_b'].reshape(1, -1, 1, 1)
    h = jax.nn.relu(h)
    return _conv1x1(h, params['pe_w2'], params['pe_b2'])


def _lin_interp(x, out_size):
    L = x.shape[-1]
    pos = jnp.arange(out_size) * ((L - 1) / (out_size - 1))
    lo = jnp.floor(pos).astype(jnp.int32)
    hi = jnp.minimum(lo + 1, L - 1)
    w = (pos - lo).astype(x.dtype)
    return x[..., lo] * (1.0 - w) + x[..., hi] * w


def _identity_kernel(x_ref, o_ref):
    o_ref[...] = x_ref[...]


def _pallas_identity(x):
    return pl.pallas_call(
        _identity_kernel,
        out_shape=jax.ShapeDtypeStruct(x.shape, x.dtype),
    )(x)


def kernel(xyz, features, params):
    xyzf = xyz.transpose(0, 2, 1)
    fps_idx = _fps(xyzf, NPOINT)
    group_idx, new_xyz = _knn_pallas(xyzf, fps_idx)

    feat_tab = features.transpose(0, 2, 1).reshape(B * N, D_IN)
    xyz_tab = jnp.concatenate(
        [xyzf, jnp.zeros((B, N, 13), jnp.float32)], axis=-1).reshape(B * N, 16)
    petab = _petab_pallas(xyz_tab, feat_tab, params)
    flat_idx = (group_idx
                + (jnp.arange(B, dtype=jnp.int32) * N)[:, None, None]).reshape(-1)
    x_tok = _sc_gather(petab, flat_idx)
    pooled = _tx_pallas(x_tok, params)
    up = _fc_interp_pallas(pooled, params)
    return (new_xyz.transpose(0, 2, 1), up)
